# Initial kernel scaffold; baseline (speedup 1.0000x reference)
import jax
import jax.numpy as jnp
from jax.experimental import pallas as pl


def kernel(beta, gamma, latent_zi, latent_zj, count, sparse_i_idx, sparse_j_idx, sample_i_idx, sample_j_idx):
    raise NotImplementedError("write your pallas kernel here")


# speedup vs baseline: 240.9975x; 240.9975x over previous
import functools

import jax
import jax.numpy as jnp
from jax import lax
from jax.experimental import pallas as pl
from jax.experimental.pallas import tpu as pltpu
from jax.experimental.pallas import tpu_sc as plsc

N_I = 50000
N_J = 50000
N_EDGES = 1600000
D = 16
S_I = 2048
S_J = 2048

NC = 2
NS = 16
NW = NC * NS
EW = N_EDGES // NW
CH = 10000
NCH = EW // CH
VPC = CH // 16
SURV_CAP = 1024
SROW = S_I // NW
EPS = 1e-6


def _sqrt_vec(x):
    xi = plsc.bitcast(x, jnp.int32)
    yi = (xi >> 1) + jnp.int32(0x1FBD1DF5)
    y = plsc.bitcast(yi, jnp.float32)
    for _ in range(3):
        y = 0.5 * (y + x / y)
    return y


def _sc_body(beta_hbm, gamma_hbm, zi_hbm, zj_hbm, count_hbm, si_hbm, sj_hbm,
             smi_hbm, smj_hbm,
             link_out, zis_out, zjs_out, betas_out, gammas_out,
             mask_i, mask_j, cbuf_i, cbuf_j, surv, sidx,
             zrow_i, zrow_j, b16, g16, c16, iv16, jv16, accv, d64r, d64s):
    wid = lax.axis_index("s") * NC + lax.axis_index("c")

    zf = jnp.zeros((16,), jnp.float32)
    zi0 = jnp.zeros((16,), jnp.int32)
    ones_f = jnp.ones((16,), jnp.float32)
    ones_i = jnp.ones((16,), jnp.int32)

    def _zero(z, _):
        mask_i[pl.ds(z * 16, 16)] = zf
        mask_j[pl.ds(z * 16, 16)] = zf
        return 0
    lax.fori_loop(0, N_I // 16, _zero, 0)

    def _zero_surv(z, _):
        surv[pl.ds(z * 16, 16)] = zi0
        return 0
    lax.fori_loop(0, SURV_CAP // 16, _zero_surv, 0)

    pltpu.sync_copy(smi_hbm, cbuf_i.at[pl.ds(0, S_I)])
    pltpu.sync_copy(smj_hbm, cbuf_j.at[pl.ds(0, S_J)])

    def _scatter_ones(k, _):
        plsc.store_scatter(mask_i, [cbuf_i[pl.ds(k * 16, 16)]], ones_f)
        plsc.store_scatter(mask_j, [cbuf_j[pl.ds(k * 16, 16)]], ones_f)
        return 0
    lax.fori_loop(0, S_I // 16, _scatter_ones, 0)

    ebase = wid * EW

    def _chunk(c, n_surv):
        base = ebase + c * CH
        pltpu.sync_copy(si_hbm.at[pl.ds(base, CH)], cbuf_i)
        pltpu.sync_copy(sj_hbm.at[pl.ds(base, CH)], cbuf_j)

        def _vec(k, ns):
            iv = cbuf_i[pl.ds(k * 16, 16)]
            jv = cbuf_j[pl.ds(k * 16, 16)]
            mi = plsc.load_gather(mask_i, [iv])
            mj = plsc.load_gather(mask_j, [jv])
            m = (mi * mj) > 0.5
            msel = jnp.where(m, ones_i, zi0)
            pos = plsc.cumsum(msel)
            tot = jnp.max(pos)
            dest = jnp.minimum(pos - 1 + ns, SURV_CAP - 1)
            eg = (base + k * 16) + lax.iota(jnp.int32, 16)
            plsc.store_scatter(surv, [dest], eg, mask=m)
            return ns + tot
        return lax.fori_loop(0, VPC, _vec, n_surv)

    n_surv = lax.fori_loop(0, NCH, _chunk, jnp.int32(0))
    n_surv = jnp.minimum(n_surv, SURV_CAP)

    nchunks = (n_surv + 15) // 16

    def _surv_chunk(c, acc):
        ssl = surv.at[pl.ds(c * 16, 16)]
        pltpu.sync_copy(si_hbm.at[ssl], iv16)
        pltpu.sync_copy(sj_hbm.at[ssl], jv16)
        pltpu.sync_copy(count_hbm.at[ssl], c16)
        pltpu.sync_copy(beta_hbm.at[iv16], b16)
        pltpu.sync_copy(gamma_hbm.at[jv16], g16)
        pltpu.sync_copy(zi_hbm.at[iv16], zrow_i)
        pltpu.sync_copy(zj_hbm.at[jv16], zrow_j)
        bv = b16[...]
        gv = g16[...]
        cv = c16[...]
        lanes = lax.iota(jnp.int32, 16)
        d2v = zf
        for d in range(D):
            dcol = jnp.full((16,), d, jnp.int32)
            av = plsc.load_gather(zrow_i, [lanes, dcol])
            bvv = plsc.load_gather(zrow_j, [lanes, dcol])
            diff = av - bvv + EPS
            d2v = d2v + diff * diff
        distv = _sqrt_vec(d2v)
        wv = jnp.where(c * 16 + lanes < n_surv, ones_f, zf)
        return acc + (bv + gv - distv) * cv * wv

    acc = lax.fori_loop(0, nchunks, _surv_chunk, zf)

    sb = wid * SROW
    pltpu.sync_copy(smi_hbm.at[pl.ds(sb, SROW)], sidx)
    pltpu.sync_copy(zi_hbm.at[sidx], d64r)
    pltpu.sync_copy(d64r, zis_out.at[pl.ds(sb, SROW)])
    pltpu.sync_copy(beta_hbm.at[sidx], d64s)
    pltpu.sync_copy(d64s, betas_out.at[pl.ds(sb, SROW)])
    pltpu.sync_copy(smj_hbm.at[pl.ds(sb, SROW)], sidx)
    pltpu.sync_copy(zj_hbm.at[sidx], d64r)
    pltpu.sync_copy(d64r, zjs_out.at[pl.ds(sb, SROW)])
    pltpu.sync_copy(gamma_hbm.at[sidx], d64s)
    pltpu.sync_copy(d64s, gammas_out.at[pl.ds(sb, SROW)])

    accv[...] = acc
    pltpu.sync_copy(accv, link_out.at[wid])


_sc_call = pl.kernel(
    _sc_body,
    out_type=[
        jax.ShapeDtypeStruct((NW, 16), jnp.float32),
        jax.ShapeDtypeStruct((S_I, D), jnp.float32),
        jax.ShapeDtypeStruct((S_J, D), jnp.float32),
        jax.ShapeDtypeStruct((S_I,), jnp.float32),
        jax.ShapeDtypeStruct((S_J,), jnp.float32),
    ],
    mesh=plsc.VectorSubcoreMesh(core_axis_name="c", subcore_axis_name="s",
                                num_cores=NC, num_subcores=NS),
    compiler_params=pltpu.CompilerParams(needs_layout_passes=False,
                                         use_tc_tiling_on_sc=False),
    scratch_types=[
        pltpu.VMEM((N_I,), jnp.float32),
        pltpu.VMEM((N_J,), jnp.float32),
        pltpu.VMEM((CH,), jnp.int32),
        pltpu.VMEM((CH,), jnp.int32),
        pltpu.VMEM((SURV_CAP,), jnp.int32),
        pltpu.VMEM((SROW,), jnp.int32),
        pltpu.VMEM((16, D), jnp.float32),
        pltpu.VMEM((16, D), jnp.float32),
        pltpu.VMEM((16,), jnp.float32),
        pltpu.VMEM((16,), jnp.float32),
        pltpu.VMEM((16,), jnp.float32),
        pltpu.VMEM((16,), jnp.int32),
        pltpu.VMEM((16,), jnp.int32),
        pltpu.VMEM((16,), jnp.float32),
        pltpu.VMEM((SROW, D), jnp.float32),
        pltpu.VMEM((SROW,), jnp.float32),
    ],
)


def _dense_body(zi_ref, zjt_ref, bcol_ref, grow_ref, out_ref):
    a = zi_ref[...]
    bt = zjt_ref[...]
    ab = lax.dot_general(a, bt, (((1,), (0,)), ((), ())),
                         preferred_element_type=jnp.float32,
                         precision=lax.Precision.HIGHEST)
    na = jnp.sum(a * a, axis=1, keepdims=True)
    sa = jnp.sum(a, axis=1, keepdims=True)
    nb = jnp.sum(bt * bt, axis=0, keepdims=True)
    sb = jnp.sum(bt, axis=0, keepdims=True)
    d2 = na + nb - 2.0 * ab + (2.0 * EPS) * (sa - sb) + (D * EPS * EPS)
    lam = bcol_ref[...] + grow_ref[...] - jnp.sqrt(jnp.maximum(d2, 0.0))
    part = jnp.sum(jnp.exp(lam))

    @pl.when(pl.program_id(0) == 0)
    def _init():
        out_ref[...] = jnp.zeros_like(out_ref)

    out_ref[...] = out_ref[...] + jnp.reshape(part, (1, 1))


_BM = 256
_dense_call = pl.pallas_call(
    _dense_body,
    grid=(S_I // _BM,),
    in_specs=[
        pl.BlockSpec((_BM, D), lambda i: (i, 0)),
        pl.BlockSpec((D, S_J), lambda i: (0, 0)),
        pl.BlockSpec((_BM, 1), lambda i: (i, 0)),
        pl.BlockSpec((1, S_J), lambda i: (0, 0)),
    ],
    out_specs=pl.BlockSpec((1, 1), lambda i: (0, 0)),
    out_shape=jax.ShapeDtypeStruct((1, 1), jnp.float32),
)


def kernel(beta, gamma, latent_zi, latent_zj, count, sparse_i_idx,
           sparse_j_idx, sample_i_idx, sample_j_idx):
    si = sparse_i_idx.astype(jnp.int32)
    sj = sparse_j_idx.astype(jnp.int32)
    smi = sample_i_idx.astype(jnp.int32)
    smj = sample_j_idx.astype(jnp.int32)

    link_parts, zi_s, zj_s, beta_s, gamma_s = _sc_call(
        beta, gamma, latent_zi, latent_zj, count, si, sj, smi, smj)

    dense = _dense_call(zi_s, zj_s.T,
                        beta_s.reshape(S_I, 1), gamma_s.reshape(1, S_J))

    return jnp.sum(link_parts) - dense[0, 0]


# speedup vs baseline: 284.9086x; 1.1822x over previous
import functools

import jax
import jax.numpy as jnp
from jax import lax
from jax.experimental import pallas as pl
from jax.experimental.pallas import tpu as pltpu
from jax.experimental.pallas import tpu_sc as plsc

N_I = 50000
N_J = 50000
N_EDGES = 1600000
D = 16
S_I = 2048
S_J = 2048

NC = 2
NS = 16
NW = NC * NS
EW = N_EDGES // NW
CH = 2000
NCH = EW // CH
VPC = CH // 16
SURV_CAP = 1024
SROW = S_I // NW
EPS = 1e-6


def _sqrt_vec(x):
    xi = plsc.bitcast(x, jnp.int32)
    yi = (xi >> 1) + jnp.int32(0x1FBD1DF5)
    y = plsc.bitcast(yi, jnp.float32)
    for _ in range(3):
        y = 0.5 * (y + x / y)
    return y


def _sc_gather_body(beta_hbm, gamma_hbm, zi_hbm, zj_hbm, smi_hbm, smj_hbm,
                    zis_out, zjs_out, betas_out, gammas_out,
                    sidx, d64r, d64s):
    wid = lax.axis_index("s") * NC + lax.axis_index("c")
    sb = wid * SROW
    pltpu.sync_copy(smi_hbm.at[pl.ds(sb, SROW)], sidx)
    pltpu.sync_copy(zi_hbm.at[sidx], d64r)
    pltpu.sync_copy(d64r, zis_out.at[pl.ds(sb, SROW)])
    pltpu.sync_copy(beta_hbm.at[sidx], d64s)
    pltpu.sync_copy(d64s, betas_out.at[pl.ds(sb, SROW)])
    pltpu.sync_copy(smj_hbm.at[pl.ds(sb, SROW)], sidx)
    pltpu.sync_copy(zj_hbm.at[sidx], d64r)
    pltpu.sync_copy(d64r, zjs_out.at[pl.ds(sb, SROW)])
    pltpu.sync_copy(gamma_hbm.at[sidx], d64s)
    pltpu.sync_copy(d64s, gammas_out.at[pl.ds(sb, SROW)])


_SC_PARAMS = dict(
    mesh=plsc.VectorSubcoreMesh(core_axis_name="c", subcore_axis_name="s",
                                num_cores=NC, num_subcores=NS),
    compiler_params=pltpu.CompilerParams(needs_layout_passes=False,
                                         use_tc_tiling_on_sc=False),
)

_sc_gather_call = pl.kernel(
    _sc_gather_body,
    out_type=[
        jax.ShapeDtypeStruct((S_I, D), jnp.float32),
        jax.ShapeDtypeStruct((S_J, D), jnp.float32),
        jax.ShapeDtypeStruct((S_I,), jnp.float32),
        jax.ShapeDtypeStruct((S_J,), jnp.float32),
    ],
    scratch_types=[
        pltpu.VMEM((SROW,), jnp.int32),
        pltpu.VMEM((SROW, D), jnp.float32),
        pltpu.VMEM((SROW,), jnp.float32),
    ],
    **_SC_PARAMS,
)


def _sc_body(beta_hbm, gamma_hbm, zi_hbm, zj_hbm, count_hbm, si_hbm, sj_hbm,
             smi_hbm, smj_hbm,
             link_out,
             mask_i, mask_j, cb_i0, cb_j0, cb_i1, cb_j1, sbuf, surv,
             ivals, jvals, cvals, bvals, gvals, zrows_i, zrows_j, accv,
             sem_e0, sem_e1, sem_c):
    wid = lax.axis_index("s") * NC + lax.axis_index("c")

    zf = jnp.zeros((16,), jnp.float32)
    zi0 = jnp.zeros((16,), jnp.int32)
    ones_f = jnp.ones((16,), jnp.float32)
    ones_i = jnp.ones((16,), jnp.int32)

    dsmp = pltpu.async_copy(smi_hbm, sbuf, sem_c)

    @plsc.parallel_loop(0, N_I // 16, unroll=8)
    def _zero(z):
        mask_i[pl.ds(z * 16, 16)] = zf
        mask_j[pl.ds(z * 16, 16)] = zf

    @plsc.parallel_loop(0, SURV_CAP // 16, unroll=4)
    def _zero_surv(z):
        surv[pl.ds(z * 16, 16)] = zi0

    dsmp.wait()

    @plsc.parallel_loop(0, S_I // 16, unroll=4)
    def _scatter_ones_i(k):
        plsc.store_scatter(mask_i, [sbuf[pl.ds(k * 16, 16)]], ones_f)

    pltpu.sync_copy(smj_hbm, sbuf)

    @plsc.parallel_loop(0, S_J // 16, unroll=4)
    def _scatter_ones_j(k):
        plsc.store_scatter(mask_j, [sbuf[pl.ds(k * 16, 16)]], ones_f)

    ebase = wid * EW
    bufs = ((cb_i0, cb_j0, sem_e0), (cb_i1, cb_j1, sem_e1))

    def _fetch(c):
        bi, bj, sem = bufs[c % 2]
        base = ebase + c * CH
        return (pltpu.async_copy(si_hbm.at[pl.ds(base, CH)], bi, sem),
                pltpu.async_copy(sj_hbm.at[pl.ds(base, CH)], bj, sem))

    ns = jnp.int32(0)
    pend = _fetch(0)
    for c in range(NCH):
        bi, bj, _ = bufs[c % 2]
        nxt = _fetch(c + 1) if c + 1 < NCH else None
        pend[0].wait()
        pend[1].wait()
        base = ebase + c * CH

        @plsc.parallel_loop(0, VPC, unroll=4, carry=ns)
        def _vec(k, ns_):
            iv = bi[pl.ds(k * 16, 16)]
            jv = bj[pl.ds(k * 16, 16)]
            mi = plsc.load_gather(mask_i, [iv])
            mj = plsc.load_gather(mask_j, [jv])
            m = (mi * mj) > 0.5
            cnt = plsc.all_reduce_population_count(m)[0]
            msel = jnp.where(m, ones_i, zi0)
            pos = plsc.cumsum(msel)
            dest = jnp.minimum(pos - 1 + ns_, SURV_CAP - 1)
            eg = (base + k * 16) + lax.iota(jnp.int32, 16)
            plsc.store_scatter(surv, [dest], eg, mask=m)
            return ns_ + cnt

        ns = _vec
        pend = nxt

    n_surv = jnp.minimum(ns, SURV_CAP)

    nblk = (n_surv + 255) // 256
    lanes = lax.iota(jnp.int32, 16)

    def _blk(blk, acc):
        b0 = blk * 256
        wave1 = []
        for c2 in range(2):
            ssl = surv.at[pl.ds(b0 + c2 * 128, 128)]
            dst = pl.ds(c2 * 128, 128)
            wave1 += [pltpu.async_copy(si_hbm.at[ssl], ivals.at[dst], sem_c),
                      pltpu.async_copy(sj_hbm.at[ssl], jvals.at[dst], sem_c),
                      pltpu.async_copy(count_hbm.at[ssl], cvals.at[dst], sem_c)]
        for dd in wave1:
            dd.wait()
        wave2 = []
        for c2 in range(2):
            isl = ivals.at[pl.ds(c2 * 128, 128)]
            jsl = jvals.at[pl.ds(c2 * 128, 128)]
            dst = pl.ds(c2 * 128, 128)
            wave2 += [pltpu.async_copy(beta_hbm.at[isl], bvals.at[dst], sem_c),
                      pltpu.async_copy(gamma_hbm.at[jsl], gvals.at[dst], sem_c),
                      pltpu.async_copy(zi_hbm.at[isl], zrows_i.at[dst], sem_c),
                      pltpu.async_copy(zj_hbm.at[jsl], zrows_j.at[dst], sem_c)]
        for dd in wave2:
            dd.wait()

        @plsc.parallel_loop(0, 16, unroll=4, carry=acc)
        def _cv(k, a):
            rows = k * 16 + lanes
            d2v = zf
            for d in range(D):
                dcol = jnp.full((16,), d, jnp.int32)
                av = plsc.load_gather(zrows_i, [rows, dcol])
                bvv = plsc.load_gather(zrows_j, [rows, dcol])
                diff = av - bvv + EPS
                d2v = d2v + diff * diff
            distv = _sqrt_vec(d2v)
            bv = bvals[pl.ds(k * 16, 16)]
            gv = gvals[pl.ds(k * 16, 16)]
            cv = cvals[pl.ds(k * 16, 16)]
            wv = jnp.where(b0 + k * 16 + lanes < n_surv, ones_f, zf)
            return a + (bv + gv - distv) * cv * wv

        return _cv

    acc = lax.fori_loop(0, nblk, _blk, zf)

    accv[...] = acc
    pltpu.sync_copy(accv, link_out.at[wid])


_sc_call = pl.kernel(
    _sc_body,
    out_type=[
        jax.ShapeDtypeStruct((NW, 16), jnp.float32),
    ],
    scratch_types=[
        pltpu.VMEM((N_I,), jnp.float32),
        pltpu.VMEM((N_J,), jnp.float32),
        pltpu.VMEM((CH,), jnp.int32),
        pltpu.VMEM((CH,), jnp.int32),
        pltpu.VMEM((CH,), jnp.int32),
        pltpu.VMEM((CH,), jnp.int32),
        pltpu.VMEM((S_I,), jnp.int32),
        pltpu.VMEM((SURV_CAP,), jnp.int32),
        pltpu.VMEM((256,), jnp.int32),
        pltpu.VMEM((256,), jnp.int32),
        pltpu.VMEM((256,), jnp.float32),
        pltpu.VMEM((256,), jnp.float32),
        pltpu.VMEM((256,), jnp.float32),
        pltpu.VMEM((256, D), jnp.float32),
        pltpu.VMEM((256, D), jnp.float32),
        pltpu.VMEM((16,), jnp.float32),
        pltpu.SemaphoreType.DMA,
        pltpu.SemaphoreType.DMA,
        pltpu.SemaphoreType.DMA,
    ],
    **_SC_PARAMS,
)


def _dense_body(zi_ref, zjt_ref, bcol_ref, grow_ref, out_ref):
    a = zi_ref[...]
    bt = zjt_ref[...]
    ab = lax.dot_general(a, bt, (((1,), (0,)), ((), ())),
                         preferred_element_type=jnp.float32,
                         precision=lax.Precision.HIGHEST)
    na = jnp.sum(a * a, axis=1, keepdims=True)
    sa = jnp.sum(a, axis=1, keepdims=True)
    nb = jnp.sum(bt * bt, axis=0, keepdims=True)
    sb = jnp.sum(bt, axis=0, keepdims=True)
    d2 = na + nb - 2.0 * ab + (2.0 * EPS) * (sa - sb) + (D * EPS * EPS)
    lam = bcol_ref[...] + grow_ref[...] - jnp.sqrt(jnp.maximum(d2, 0.0))
    part = jnp.sum(jnp.exp(lam))

    @pl.when(pl.program_id(0) == 0)
    def _init():
        out_ref[...] = jnp.zeros_like(out_ref)

    out_ref[...] = out_ref[...] + jnp.reshape(part, (1, 1))


_BM = 256
_dense_call = pl.pallas_call(
    _dense_body,
    grid=(S_I // _BM,),
    in_specs=[
        pl.BlockSpec((_BM, D), lambda i: (i, 0)),
        pl.BlockSpec((D, S_J), lambda i: (0, 0)),
        pl.BlockSpec((_BM, 1), lambda i: (i, 0)),
        pl.BlockSpec((1, S_J), lambda i: (0, 0)),
    ],
    out_specs=pl.BlockSpec((1, 1), lambda i: (0, 0)),
    out_shape=jax.ShapeDtypeStruct((1, 1), jnp.float32),
)


def kernel(beta, gamma, latent_zi, latent_zj, count, sparse_i_idx,
           sparse_j_idx, sample_i_idx, sample_j_idx):
    si = sparse_i_idx.astype(jnp.int32)
    sj = sparse_j_idx.astype(jnp.int32)
    smi = sample_i_idx.astype(jnp.int32)
    smj = sample_j_idx.astype(jnp.int32)

    zi_s, zj_s, beta_s, gamma_s = _sc_gather_call(
        beta, gamma, latent_zi, latent_zj, smi, smj)

    (link_parts,) = _sc_call(
        beta, gamma, latent_zi, latent_zj, count, si, sj, smi, smj)

    dense = _dense_call(zi_s, zj_s.T,
                        beta_s.reshape(S_I, 1), gamma_s.reshape(1, S_J))

    return jnp.sum(link_parts) - dense[0, 0]


# speedup vs baseline: 288.1931x; 1.0115x over previous
import functools

import jax
import jax.numpy as jnp
from jax import lax
from jax.experimental import pallas as pl
from jax.experimental.pallas import tpu as pltpu
from jax.experimental.pallas import tpu_sc as plsc

N_I = 50000
N_J = 50000
N_EDGES = 1600000
D = 16
S_I = 2048
S_J = 2048

NC = 2
NS = 16
NW = NC * NS
EW = N_EDGES // NW
CH = 10000
NCH = EW // CH
VPC = CH // 16
MW = 1568
SURV_CAP = 1024
SROW = S_I // NW
EPS = 1e-6


def _sqrt_vec(x):
    xi = plsc.bitcast(x, jnp.int32)
    yi = (xi >> 1) + jnp.int32(0x1FBD1DF5)
    y = plsc.bitcast(yi, jnp.float32)
    for _ in range(3):
        y = 0.5 * (y + x / y)
    return y


def _sc_gather_body(beta_hbm, gamma_hbm, zi_hbm, zj_hbm, smi_hbm, smj_hbm,
                    zis_out, zjs_out, betas_out, gammas_out,
                    sidx, d64r, d64s):
    wid = lax.axis_index("s") * NC + lax.axis_index("c")
    sb = wid * SROW
    pltpu.sync_copy(smi_hbm.at[pl.ds(sb, SROW)], sidx)
    pltpu.sync_copy(zi_hbm.at[sidx], d64r)
    pltpu.sync_copy(d64r, zis_out.at[pl.ds(sb, SROW)])
    pltpu.sync_copy(beta_hbm.at[sidx], d64s)
    pltpu.sync_copy(d64s, betas_out.at[pl.ds(sb, SROW)])
    pltpu.sync_copy(smj_hbm.at[pl.ds(sb, SROW)], sidx)
    pltpu.sync_copy(zj_hbm.at[sidx], d64r)
    pltpu.sync_copy(d64r, zjs_out.at[pl.ds(sb, SROW)])
    pltpu.sync_copy(gamma_hbm.at[sidx], d64s)
    pltpu.sync_copy(d64s, gammas_out.at[pl.ds(sb, SROW)])


def _sc_params():
    return dict(
        mesh=plsc.VectorSubcoreMesh(core_axis_name="c", subcore_axis_name="s",
                                    num_cores=NC, num_subcores=NS),
        compiler_params=pltpu.CompilerParams(needs_layout_passes=False,
                                             use_tc_tiling_on_sc=False),
    )


@functools.cache
def _sc_gather_call():
    return pl.kernel(
        _sc_gather_body,
        out_type=[
            jax.ShapeDtypeStruct((S_I, D), jnp.float32),
            jax.ShapeDtypeStruct((S_J, D), jnp.float32),
            jax.ShapeDtypeStruct((S_I,), jnp.float32),
            jax.ShapeDtypeStruct((S_J,), jnp.float32),
        ],
        scratch_types=[
            pltpu.VMEM((SROW,), jnp.int32),
            pltpu.VMEM((SROW, D), jnp.float32),
            pltpu.VMEM((SROW,), jnp.float32),
        ],
        **_sc_params(),
    )


def _sc_body(beta_hbm, gamma_hbm, zi_hbm, zj_hbm, count_hbm, si_hbm, sj_hbm,
             smi_hbm, smj_hbm,
             link_out,
             mask_i, mask_j, cb_i0, cb_j0, cb_i1, cb_j1, sbuf, surv,
             ivals, jvals, cvals, bvals, gvals, zrows_i, zrows_j, accv,
             sem_e0, sem_e1, sem_c):
    wid = lax.axis_index("s") * NC + lax.axis_index("c")

    zf = jnp.zeros((16,), jnp.float32)
    zi0 = jnp.zeros((16,), jnp.int32)
    ones_f = jnp.ones((16,), jnp.float32)
    ones_i = jnp.ones((16,), jnp.int32)

    dsmp = pltpu.async_copy(smi_hbm, sbuf, sem_c)

    @plsc.parallel_loop(0, MW // 16, unroll=8)
    def _zero(z):
        mask_i[pl.ds(z * 16, 16)] = zi0
        mask_j[pl.ds(z * 16, 16)] = zi0

    @plsc.parallel_loop(0, SURV_CAP // 16, unroll=4)
    def _zero_surv(z):
        surv[pl.ds(z * 16, 16)] = zi0

    dsmp.wait()

    @plsc.parallel_loop(0, S_I // 16, unroll=4)
    def _scatter_ones_i(k):
        iv = sbuf[pl.ds(k * 16, 16)]
        plsc.addupdate_scatter(mask_i, [iv >> 5],
                               lax.shift_left(ones_i, iv & 31))

    pltpu.sync_copy(smj_hbm, sbuf)

    @plsc.parallel_loop(0, S_J // 16, unroll=4)
    def _scatter_ones_j(k):
        jv = sbuf[pl.ds(k * 16, 16)]
        plsc.addupdate_scatter(mask_j, [jv >> 5],
                               lax.shift_left(ones_i, jv & 31))

    ebase = wid * EW
    bufs = ((cb_i0, cb_j0, sem_e0), (cb_i1, cb_j1, sem_e1))

    def _fetch(c):
        bi, bj, sem = bufs[c % 2]
        base = ebase + c * CH
        return (pltpu.async_copy(si_hbm.at[pl.ds(base, CH)], bi, sem),
                pltpu.async_copy(sj_hbm.at[pl.ds(base, CH)], bj, sem))

    ns = jnp.int32(0)
    pend = _fetch(0)
    for c in range(NCH):
        bi, bj, _ = bufs[c % 2]
        nxt = _fetch(c + 1) if c + 1 < NCH else None
        pend[0].wait()
        pend[1].wait()
        base = ebase + c * CH

        @plsc.parallel_loop(0, VPC, unroll=4, carry=ns)
        def _vec(k, ns_):
            iv = bi[pl.ds(k * 16, 16)]
            jv = bj[pl.ds(k * 16, 16)]
            wi = plsc.load_gather(mask_i, [iv >> 5])
            wj = plsc.load_gather(mask_j, [jv >> 5])
            ti = lax.shift_right_logical(wi, iv & 31)
            tj = lax.shift_right_logical(wj, jv & 31)
            m = (ti & tj & 1) > 0
            cnt = plsc.all_reduce_population_count(m)[0]
            msel = jnp.where(m, ones_i, zi0)
            pos = plsc.cumsum(msel)
            dest = jnp.minimum(pos - 1 + ns_, SURV_CAP - 1)
            eg = (base + k * 16) + lax.iota(jnp.int32, 16)
            plsc.store_scatter(surv, [dest], eg, mask=m)
            return ns_ + cnt

        ns = _vec
        pend = nxt

    n_surv = jnp.minimum(ns, SURV_CAP)

    nblk = (n_surv + 255) // 256
    lanes = lax.iota(jnp.int32, 16)

    def _blk(blk, acc):
        b0 = blk * 256
        wave1 = []
        for c2 in range(2):
            ssl = surv.at[pl.ds(b0 + c2 * 128, 128)]
            dst = pl.ds(c2 * 128, 128)
            wave1 += [pltpu.async_copy(si_hbm.at[ssl], ivals.at[dst], sem_c),
                      pltpu.async_copy(sj_hbm.at[ssl], jvals.at[dst], sem_c),
                      pltpu.async_copy(count_hbm.at[ssl], cvals.at[dst], sem_c)]
        for dd in wave1:
            dd.wait()
        wave2 = []
        for c2 in range(2):
            isl = ivals.at[pl.ds(c2 * 128, 128)]
            jsl = jvals.at[pl.ds(c2 * 128, 128)]
            dst = pl.ds(c2 * 128, 128)
            wave2 += [pltpu.async_copy(beta_hbm.at[isl], bvals.at[dst], sem_c),
                      pltpu.async_copy(gamma_hbm.at[jsl], gvals.at[dst], sem_c),
                      pltpu.async_copy(zi_hbm.at[isl], zrows_i.at[dst], sem_c),
                      pltpu.async_copy(zj_hbm.at[jsl], zrows_j.at[dst], sem_c)]
        for dd in wave2:
            dd.wait()

        @plsc.parallel_loop(0, 16, unroll=4, carry=acc)
        def _cv(k, a):
            rows = k * 16 + lanes
            d2v = zf
            for d in range(D):
                dcol = jnp.full((16,), d, jnp.int32)
                av = plsc.load_gather(zrows_i, [rows, dcol])
                bvv = plsc.load_gather(zrows_j, [rows, dcol])
                diff = av - bvv + EPS
                d2v = d2v + diff * diff
            distv = _sqrt_vec(d2v)
            bv = bvals[pl.ds(k * 16, 16)]
            gv = gvals[pl.ds(k * 16, 16)]
            cv = cvals[pl.ds(k * 16, 16)]
            wv = jnp.where(b0 + k * 16 + lanes < n_surv, ones_f, zf)
            return a + (bv + gv - distv) * cv * wv

        return _cv

    acc = lax.fori_loop(0, nblk, _blk, zf)

    accv[...] = acc
    pltpu.sync_copy(accv, link_out.at[wid])


@functools.cache
def _sc_call():
    return pl.kernel(
        _sc_body,
        out_type=[
            jax.ShapeDtypeStruct((NW, 16), jnp.float32),
        ],
        scratch_types=[
            pltpu.VMEM((MW,), jnp.int32),
            pltpu.VMEM((MW,), jnp.int32),
            pltpu.VMEM((CH,), jnp.int32),
            pltpu.VMEM((CH,), jnp.int32),
            pltpu.VMEM((CH,), jnp.int32),
            pltpu.VMEM((CH,), jnp.int32),
            pltpu.VMEM((S_I,), jnp.int32),
            pltpu.VMEM((SURV_CAP,), jnp.int32),
            pltpu.VMEM((256,), jnp.int32),
            pltpu.VMEM((256,), jnp.int32),
            pltpu.VMEM((256,), jnp.float32),
            pltpu.VMEM((256,), jnp.float32),
            pltpu.VMEM((256,), jnp.float32),
            pltpu.VMEM((256, D), jnp.float32),
            pltpu.VMEM((256, D), jnp.float32),
            pltpu.VMEM((16,), jnp.float32),
            pltpu.SemaphoreType.DMA,
            pltpu.SemaphoreType.DMA,
            pltpu.SemaphoreType.DMA,
        ],
        **_sc_params(),
    )


def _dense_body(zi_ref, zjt_ref, bcol_ref, grow_ref, out_ref):
    a = zi_ref[...]
    bt = zjt_ref[...]
    ab = lax.dot_general(a, bt, (((1,), (0,)), ((), ())),
                         preferred_element_type=jnp.float32,
                         precision=lax.Precision.HIGHEST)
    na = jnp.sum(a * a, axis=1, keepdims=True)
    sa = jnp.sum(a, axis=1, keepdims=True)
    nb = jnp.sum(bt * bt, axis=0, keepdims=True)
    sb = jnp.sum(bt, axis=0, keepdims=True)
    d2 = na + nb - 2.0 * ab + (2.0 * EPS) * (sa - sb) + (D * EPS * EPS)
    lam = bcol_ref[...] + grow_ref[...] - jnp.sqrt(jnp.maximum(d2, 0.0))
    part = jnp.sum(jnp.exp(lam))

    @pl.when(pl.program_id(0) == 0)
    def _init():
        out_ref[...] = jnp.zeros_like(out_ref)

    out_ref[...] = out_ref[...] + jnp.reshape(part, (1, 1))


_BM = 256
_dense_call = pl.pallas_call(
    _dense_body,
    grid=(S_I // _BM,),
    in_specs=[
        pl.BlockSpec((_BM, D), lambda i: (i, 0)),
        pl.BlockSpec((D, S_J), lambda i: (0, 0)),
        pl.BlockSpec((_BM, 1), lambda i: (i, 0)),
        pl.BlockSpec((1, S_J), lambda i: (0, 0)),
    ],
    out_specs=pl.BlockSpec((1, 1), lambda i: (0, 0)),
    out_shape=jax.ShapeDtypeStruct((1, 1), jnp.float32),
)


def kernel(beta, gamma, latent_zi, latent_zj, count, sparse_i_idx,
           sparse_j_idx, sample_i_idx, sample_j_idx):
    si = sparse_i_idx.astype(jnp.int32)
    sj = sparse_j_idx.astype(jnp.int32)
    smi = sample_i_idx.astype(jnp.int32)
    smj = sample_j_idx.astype(jnp.int32)

    zi_s, zj_s, beta_s, gamma_s = _sc_gather_call()(
        beta, gamma, latent_zi, latent_zj, smi, smj)

    (link_parts,) = _sc_call()(
        beta, gamma, latent_zi, latent_zj, count, si, sj, smi, smj)

    dense = _dense_call(zi_s, zj_s.T,
                        beta_s.reshape(S_I, 1), gamma_s.reshape(1, S_J))

    return jnp.sum(link_parts) - dense[0, 0]
